# baseline (speedup 1.0000x reference)
import functools

import jax
import jax.numpy as jnp
from jax import lax
from jax.experimental import pallas as pl
from jax.experimental.pallas import tpu as pltpu
from jax.experimental.pallas import tpu_sc as plsc

N = 5000
E = 80000
H = 128
NRBF = 64
MAXZ = 128
CUT_UP = 5.0

NPAD = 5120
NC = 2
NS = 16
NCH = 9

NSLICE = 4
ESL = E // NSLICE
EB = 800
NB_SL = ESL // EB

EPAD_G = 81920
EPT_G = EPAD_G // (NC * NS)
CHUNK_G = 128

EPT_S = ESL // NS
CHUNK_S = 125
NCH_S = EPT_S // CHUNK_S
ROWS_PT = NPAD // NS
ZROWS = 64

NBLK = 512

SC_CH_BASE = (0, 5)
SC_CH_CNT = (5, 4)


def _gather_body(src_hbm, dst_hbm, z_hbm, zsrc_hbm, zdst_hbm,
                 eslab, oslab, sem):
    cid = lax.axis_index("c")
    sid = lax.axis_index("s")
    wid = cid * NS + sid
    base = wid * EPT_G
    nch = EPT_G // CHUNK_G

    def do_half(e_hbm, o_hbm):
        pltpu.sync_copy(e_hbm.at[wid], eslab)
        descs = []
        for j in range(nch):
            d = pltpu.make_async_copy(
                z_hbm.at[eslab.at[j]],
                oslab.at[pl.ds(j * CHUNK_G, CHUNK_G)], sem)
            d.start()
            descs.append(d)
        for d in descs:
            d.wait()
        pltpu.sync_copy(oslab, o_hbm.at[pl.ds(base, EPT_G)])

    do_half(src_hbm, zsrc_hbm)
    do_half(dst_hbm, zdst_hbm)


def _sc_gather(src_flat, dst_flat, z):
    mesh = plsc.VectorSubcoreMesh(core_axis_name="c", subcore_axis_name="s")
    nw = NC * NS
    src_r = src_flat.reshape(nw, EPT_G // CHUNK_G, CHUNK_G)
    dst_r = dst_flat.reshape(nw, EPT_G // CHUNK_G, CHUNK_G)
    return pl.kernel(
        _gather_body,
        out_type=[jax.ShapeDtypeStruct((EPAD_G,), jnp.int32),
                  jax.ShapeDtypeStruct((EPAD_G,), jnp.int32)],
        mesh=mesh,
        scratch_types=[pltpu.VMEM((EPT_G // CHUNK_G, CHUNK_G), jnp.int32),
                       pltpu.VMEM((EPT_G,), jnp.int32),
                       pltpu.SemaphoreType.DMA],
    )(src_r, dst_r, z)


def _prep_body(emb_ref, wlt_ref, wrt_ref, tcat_ref):
    f32 = jnp.float32
    tl = jnp.dot(emb_ref[...], wlt_ref[...], preferred_element_type=f32)
    tr = jnp.dot(emb_ref[...], wrt_ref[...], preferred_element_type=f32)
    tcat_ref[...] = jnp.concatenate([tl, tr], axis=0)


def _tc_prep(emb, wlt, wrt):
    return pl.pallas_call(
        _prep_body,
        out_shape=jax.ShapeDtypeStruct((2 * H, H), jnp.float32),
    )(emb, wlt, wrt)


def _edge_body(attr_ref, misc_ref, w123t_ref, b123_ref, tcat_ref, embb_ref,
               out_ref):
    f32 = jnp.float32
    mt = misc_ref[0].T
    c = mt[:, 0:1]
    vx = mt[:, 1:2]
    vy = mt[:, 2:3]
    vz = mt[:, 3:4]
    zs = mt[:, 4:5]
    zd = mt[:, 5:6]
    ioz = lax.broadcasted_iota(jnp.int32, (EB, MAXZ), 1).astype(f32)
    oh = jnp.concatenate([(zs == ioz), (zd == ioz)], axis=1).astype(f32)
    zij = (jnp.dot(oh, tcat_ref[...], preferred_element_type=f32)
           + embb_ref[...]) * c

    za = attr_ref[...]
    we = jnp.dot(za, w123t_ref[...], preferred_element_type=f32) \
        + b123_ref[...]
    m1 = zij * we[:, :H]
    m2 = zij * we[:, H:2 * H]
    m3 = zij * we[:, 2 * H:]
    r3 = (vx * vx + vy * vy + vz * vz) * (1.0 / 3.0)

    out_ref[0] = m1[:, None, :]
    out_ref[1] = (m2 * vx)[:, None, :]
    out_ref[2] = (m2 * vy)[:, None, :]
    out_ref[3] = (m2 * vz)[:, None, :]
    out_ref[4] = (m3 * (vx * vx - r3))[:, None, :]
    out_ref[5] = (m3 * (vy * vy - r3))[:, None, :]
    out_ref[6] = (m3 * (vx * vy))[:, None, :]
    out_ref[7] = (m3 * (vx * vz))[:, None, :]
    out_ref[8] = (m3 * (vy * vz))[:, None, :]


def _tc_edge(s, attr, misc, w123t, b123, tcat, embb):
    base = s * NB_SL
    full2 = lambda shape: pl.BlockSpec(shape, lambda i: (0, 0))
    return pl.pallas_call(
        _edge_body,
        grid=(NB_SL,),
        in_specs=[
            pl.BlockSpec((EB, NRBF), lambda i: (base + i, 0)),
            pl.BlockSpec((1, 8, EB), lambda i: (base + i, 0, 0)),
            full2((NRBF, 3 * H)), full2((1, 3 * H)),
            full2((2 * H, H)), full2((1, H)),
        ],
        out_specs=pl.BlockSpec((NCH, EB, 1, H), lambda i: (0, i, 0, 0)),
        out_shape=jax.ShapeDtypeStruct((NCH, ESL, 1, H), jnp.float32),
    )(attr, misc, w123t, b123, tcat, embb)


def _scatter_body(p_hbm, seg_hbm, zeros_hbm, out_hbm, segv, pbuf, zbuf, acc,
                  sem_in, sem_sc):
    cid = lax.axis_index("c")
    sid = lax.axis_index("s")
    ebase = sid * EPT_S
    rbase = sid * ROWS_PT
    pltpu.sync_copy(seg_hbm.at[sid], segv)
    pltpu.sync_copy(zeros_hbm, zbuf)

    for sc in range(NC):
        @pl.when(cid == sc)
        def _():
            for ch in range(SC_CH_CNT[sc]):
                chg = SC_CH_BASE[sc] + ch
                for k in range(ROWS_PT // ZROWS):
                    pltpu.sync_copy(
                        zbuf, acc.at[pl.ds(rbase + k * ZROWS, ZROWS)])
                plsc.subcore_barrier()

                in_d = [None] * NCH_S
                sc_d = [None] * NCH_S

                def fire_in(j):
                    d = pltpu.make_async_copy(
                        p_hbm.at[chg, pl.ds(ebase + j * CHUNK_S, CHUNK_S)],
                        pbuf.at[j % 2], sem_in)
                    d.start()
                    in_d[j] = d

                fire_in(0)
                for j in range(NCH_S):
                    if j >= 1:
                        sc_d[j - 1].wait()
                    if j + 1 < NCH_S:
                        fire_in(j + 1)
                    in_d[j].wait()
                    d = pltpu.make_async_copy(
                        pbuf.at[j % 2],
                        acc.at[segv.at[j, 0, pl.ds(0, CHUNK_S)]], sem_sc)
                    d.start(add=True)
                    sc_d[j] = d
                sc_d[NCH_S - 1].wait()
                plsc.subcore_barrier()
                pltpu.sync_copy(acc.at[pl.ds(rbase, ROWS_PT), 0],
                                out_hbm.at[chg, pl.ds(rbase, ROWS_PT)])
                plsc.subcore_barrier()


def _sc_scatter(payload, seg_r):
    mesh = plsc.VectorSubcoreMesh(core_axis_name="c", subcore_axis_name="s")
    zeros = jnp.zeros((ZROWS, 1, H), jnp.float32)
    return pl.kernel(
        _scatter_body,
        out_type=jax.ShapeDtypeStruct((NCH, NPAD, H), jnp.float32),
        mesh=mesh,
        scratch_types=[pltpu.VMEM((NCH_S, 1, CHUNK_G), jnp.int32),
                       pltpu.VMEM((2, CHUNK_S, 1, H), jnp.float32),
                       pltpu.VMEM((ZROWS, 1, H), jnp.float32),
                       pltpu.VMEM_SHARED((NPAD, 1, H), jnp.float32),
                       pltpu.SemaphoreType.DMA,
                       pltpu.SemaphoreType.DMA],
    )(payload, seg_r, zeros)


def _node_body(*refs):
    (*a_refs, lt0t_ref, lt1t_ref, lt2t_ref, ls1wt_ref, ls1b_ref,
     ls2wt_ref, ls2b_ref, lng_ref, lnb_ref, out_ref) = refs
    f32 = jnp.float32
    dot = functools.partial(jnp.dot, preferred_element_type=f32)

    def acc(k):
        v = a_refs[0][k]
        for a in a_refs[1:]:
            v = v + a[k]
        return v

    i_ = acc(0)
    ax = acc(1)
    ay = acc(2)
    az = acc(3)
    dxx = acc(4)
    dyy = acc(5)
    sxy = acc(6)
    sxz = acc(7)
    syz = acc(8)
    dzz = -dxx - dyy

    norm = (3.0 * i_ * i_
            + 2.0 * (ax * ax + ay * ay + az * az)
            + dxx * dxx + dyy * dyy + dzz * dzz
            + 2.0 * (sxy * sxy + sxz * sxz + syz * syz))
    mu = jnp.mean(norm, axis=1, keepdims=True)
    var = jnp.mean((norm - mu) ** 2, axis=1, keepdims=True)
    nrm = (norm - mu) * lax.rsqrt(var + 1e-5) * lng_ref[...] + lnb_ref[...]

    h1 = dot(nrm, ls1wt_ref[...]) + ls1b_ref[...]
    h1 = h1 * jax.nn.sigmoid(h1)
    h2 = dot(h1, ls2wt_ref[...]) + ls2b_ref[...]
    h2 = h2 * jax.nn.sigmoid(h2)
    n0 = h2[:, :H]
    n1 = h2[:, H:2 * H]
    n2 = h2[:, 2 * H:]

    lt2t = lt2t_ref[...]
    i2 = dot(i_, lt0t_ref[...])
    axp = dot(ax, lt1t_ref[...])
    ayp = dot(ay, lt1t_ref[...])
    azp = dot(az, lt1t_ref[...])
    dxxp = dot(dxx, lt2t)
    dyyp = dot(dyy, lt2t)
    dzzp = -dxxp - dyyp
    sxyp = dot(sxy, lt2t)
    sxzp = dot(sxz, lt2t)
    syzp = dot(syz, lt2t)

    diag = n0 * i2
    out_ref[0] = diag + n2 * dxxp
    out_ref[1] = -n1 * azp + n2 * sxyp
    out_ref[2] = n1 * ayp + n2 * sxzp
    out_ref[3] = n1 * azp + n2 * sxyp
    out_ref[4] = diag + n2 * dyyp
    out_ref[5] = -n1 * axp + n2 * syzp
    out_ref[6] = -n1 * ayp + n2 * sxzp
    out_ref[7] = n1 * axp + n2 * syzp
    out_ref[8] = diag + n2 * dzzp


def _tc_node(accs, lt0t, lt1t, lt2t, ls1wt, ls1br, ls2wt, ls2br,
             lngr, lnbr):
    full2 = lambda shape: pl.BlockSpec(shape, lambda i: (0, 0))
    aspec = pl.BlockSpec((NCH, NBLK, H), lambda i: (0, i, 0))
    return pl.pallas_call(
        _node_body,
        grid=(NPAD // NBLK,),
        in_specs=[aspec] * NSLICE + [
            full2((H, H)), full2((H, H)), full2((H, H)),
            full2((H, 2 * H)), full2((1, 2 * H)),
            full2((2 * H, 3 * H)), full2((1, 3 * H)),
            full2((1, H)), full2((1, H)),
        ],
        out_specs=pl.BlockSpec((9, NBLK, H), lambda i: (0, i, 0)),
        out_shape=jax.ShapeDtypeStruct((9, NPAD, H), jnp.float32),
    )(*accs, lt0t, lt1t, lt2t, ls1wt, ls1br, ls2wt, ls2br, lngr, lnbr)


def kernel(z, edge_index, edge_weight, edge_vec_norm, edge_attr,
           W1, b1, W2, b2, W3, b3, emb, emb2_W, emb2_b,
           lt0, lt1, lt2, ls1_W, ls1_b, ls2_W, ls2_b, ln_g, ln_b):
    f32 = jnp.float32
    i32 = jnp.int32
    z = z.astype(i32)
    ei = edge_index.astype(i32)
    pad_g = EPAD_G - E

    spread = jnp.arange(pad_g, dtype=i32) % N
    src_flat = jnp.concatenate([ei[0], spread])
    dst_flat = jnp.concatenate([ei[1], spread])
    z_pad = jnp.concatenate([z, jnp.zeros((NPAD - N,), i32)])

    seg4 = ei[0].reshape(NSLICE, NS, NCH_S, CHUNK_S)
    npadlanes = CHUNK_G - CHUNK_S
    dummy = N + (jnp.arange(NSLICE * NS * NCH_S * npadlanes, dtype=i32)
                 % (NPAD - N))
    seg_r = jnp.concatenate(
        [seg4, dummy.reshape(NSLICE, NS, NCH_S, npadlanes)],
        axis=3).reshape(NSLICE, NS, NCH_S, 1, CHUNK_G)

    w = edge_weight.astype(f32)
    cutoff = 0.5 * (jnp.cos(w * (jnp.pi / CUT_UP)) + 1.0)
    cutoff = cutoff * (w < CUT_UP).astype(f32)
    ev = edge_vec_norm.astype(f32)

    zsrc, zdst = _sc_gather(src_flat, dst_flat, z_pad)

    zero_e = jnp.zeros((E,), f32)
    misc = jnp.stack([cutoff, ev[:, 0], ev[:, 1], ev[:, 2],
                      zsrc[:E].astype(f32), zdst[:E].astype(f32),
                      zero_e, zero_e]).reshape(8, E // EB, EB) \
        .transpose(1, 0, 2)

    w123t = jnp.concatenate([W1.T, W2.T, W3.T], axis=1).astype(f32)
    b123 = jnp.concatenate([b1, b2, b3]).reshape(1, 3 * H).astype(f32)
    tcat = _tc_prep(emb.astype(f32), emb2_W[:, :H].T.astype(f32),
                    emb2_W[:, H:].T.astype(f32))
    embb = emb2_b.reshape(1, H)

    attr = edge_attr.astype(f32)
    accs = []
    for s in range(NSLICE):
        payload = _tc_edge(s, attr, misc, w123t, b123, tcat, embb)
        accs.append(_sc_scatter(payload, seg_r[s]))

    perm = (jnp.arange(3 * H) % 3) * H + (jnp.arange(3 * H) // 3)
    inv = jnp.argsort(perm)
    ls2_wg = ls2_W[inv]
    ls2_bg = ls2_b[inv]

    out9 = _tc_node(
        accs,
        lt0.T.astype(f32), lt1.T.astype(f32), lt2.T.astype(f32),
        ls1_W.T.astype(f32), ls1_b.reshape(1, 2 * H),
        ls2_wg.T.astype(f32), ls2_bg.reshape(1, 3 * H),
        ln_g.reshape(1, H), ln_b.reshape(1, H))

    return out9[:, :N].transpose(1, 2, 0).reshape(N, H, 3, 3)


# speedup vs baseline: 1.0222x; 1.0222x over previous
import functools

import jax
import jax.numpy as jnp
from jax import lax
from jax.experimental import pallas as pl
from jax.experimental.pallas import tpu as pltpu
from jax.experimental.pallas import tpu_sc as plsc

N = 5000
E = 80000
H = 128
NRBF = 64
MAXZ = 128
CUT_UP = 5.0

NPAD = 5120
NC = 2
NS = 16
NCH = 9

NSLICE = 2
ESL = E // NSLICE
EB = 800
NB_SL = ESL // EB

EPAD_G = 81920
EPT_G = EPAD_G // (NC * NS)
CHUNK_G = 128

EPT_S = ESL // NS
CHUNK_S = 125
NCH_S = EPT_S // CHUNK_S
ROWS_PT = NPAD // NS
ZROWS = 64

NBLK = 512

SC_CH_BASE = (0, 5)
SC_CH_CNT = (5, 4)


def _gather_body(src_hbm, dst_hbm, z_hbm, zsrc_hbm, zdst_hbm,
                 eslab, oslab, sem):
    cid = lax.axis_index("c")
    sid = lax.axis_index("s")
    wid = cid * NS + sid
    base = wid * EPT_G
    nch = EPT_G // CHUNK_G

    def do_half(e_hbm, o_hbm):
        pltpu.sync_copy(e_hbm.at[wid], eslab)
        descs = []
        for j in range(nch):
            d = pltpu.make_async_copy(
                z_hbm.at[eslab.at[j]],
                oslab.at[pl.ds(j * CHUNK_G, CHUNK_G)], sem)
            d.start()
            descs.append(d)
        for d in descs:
            d.wait()
        pltpu.sync_copy(oslab, o_hbm.at[pl.ds(base, EPT_G)])

    do_half(src_hbm, zsrc_hbm)
    do_half(dst_hbm, zdst_hbm)


def _sc_gather(src_flat, dst_flat, z):
    mesh = plsc.VectorSubcoreMesh(core_axis_name="c", subcore_axis_name="s")
    nw = NC * NS
    src_r = src_flat.reshape(nw, EPT_G // CHUNK_G, CHUNK_G)
    dst_r = dst_flat.reshape(nw, EPT_G // CHUNK_G, CHUNK_G)
    return pl.kernel(
        _gather_body,
        out_type=[jax.ShapeDtypeStruct((EPAD_G,), jnp.int32),
                  jax.ShapeDtypeStruct((EPAD_G,), jnp.int32)],
        mesh=mesh,
        scratch_types=[pltpu.VMEM((EPT_G // CHUNK_G, CHUNK_G), jnp.int32),
                       pltpu.VMEM((EPT_G,), jnp.int32),
                       pltpu.SemaphoreType.DMA],
    )(src_r, dst_r, z)


def _prep_body(emb_ref, wlt_ref, wrt_ref, tcat_ref):
    f32 = jnp.float32
    tl = jnp.dot(emb_ref[...], wlt_ref[...], preferred_element_type=f32)
    tr = jnp.dot(emb_ref[...], wrt_ref[...], preferred_element_type=f32)
    tcat_ref[...] = jnp.concatenate([tl, tr], axis=0)


def _tc_prep(emb, wlt, wrt):
    return pl.pallas_call(
        _prep_body,
        out_shape=jax.ShapeDtypeStruct((2 * H, H), jnp.float32),
    )(emb, wlt, wrt)


def _edge_body(attr_ref, misc_ref, w123t_ref, b123_ref, tcat_ref, embb_ref,
               out_ref):
    f32 = jnp.float32
    mt = misc_ref[0].T
    c = mt[:, 0:1]
    vx = mt[:, 1:2]
    vy = mt[:, 2:3]
    vz = mt[:, 3:4]
    zs = mt[:, 4:5]
    zd = mt[:, 5:6]
    ioz = lax.broadcasted_iota(jnp.int32, (EB, MAXZ), 1).astype(f32)
    oh = jnp.concatenate([(zs == ioz), (zd == ioz)], axis=1).astype(f32)
    zij = (jnp.dot(oh, tcat_ref[...], preferred_element_type=f32)
           + embb_ref[...]) * c

    za = attr_ref[...]
    we = jnp.dot(za, w123t_ref[...], preferred_element_type=f32) \
        + b123_ref[...]
    m1 = zij * we[:, :H]
    m2 = zij * we[:, H:2 * H]
    m3 = zij * we[:, 2 * H:]
    r3 = (vx * vx + vy * vy + vz * vz) * (1.0 / 3.0)

    out_ref[0] = m1[:, None, :]
    out_ref[1] = (m2 * vx)[:, None, :]
    out_ref[2] = (m2 * vy)[:, None, :]
    out_ref[3] = (m2 * vz)[:, None, :]
    out_ref[4] = (m3 * (vx * vx - r3))[:, None, :]
    out_ref[5] = (m3 * (vy * vy - r3))[:, None, :]
    out_ref[6] = (m3 * (vx * vy))[:, None, :]
    out_ref[7] = (m3 * (vx * vz))[:, None, :]
    out_ref[8] = (m3 * (vy * vz))[:, None, :]


def _tc_edge(s, attr, misc, w123t, b123, tcat, embb):
    base = s * NB_SL
    full2 = lambda shape: pl.BlockSpec(shape, lambda i: (0, 0))
    return pl.pallas_call(
        _edge_body,
        grid=(NB_SL,),
        in_specs=[
            pl.BlockSpec((EB, NRBF), lambda i: (base + i, 0)),
            pl.BlockSpec((1, 8, EB), lambda i: (base + i, 0, 0)),
            full2((NRBF, 3 * H)), full2((1, 3 * H)),
            full2((2 * H, H)), full2((1, H)),
        ],
        out_specs=pl.BlockSpec((NCH, EB, 1, H), lambda i: (0, i, 0, 0)),
        out_shape=jax.ShapeDtypeStruct((NCH, ESL, 1, H), jnp.float32),
    )(attr, misc, w123t, b123, tcat, embb)


def _scatter_body(p_hbm, seg_hbm, zeros_hbm, out_hbm, segv, pbuf, zbuf, acc,
                  sem_in, sem_sc):
    cid = lax.axis_index("c")
    sid = lax.axis_index("s")
    ebase = sid * EPT_S
    rbase = sid * ROWS_PT
    pltpu.sync_copy(seg_hbm.at[sid], segv)
    pltpu.sync_copy(zeros_hbm, zbuf)

    for sc in range(NC):
        @pl.when(cid == sc)
        def _():
            for ch in range(SC_CH_CNT[sc]):
                chg = SC_CH_BASE[sc] + ch
                for k in range(ROWS_PT // ZROWS):
                    pltpu.sync_copy(
                        zbuf, acc.at[pl.ds(rbase + k * ZROWS, ZROWS)])
                plsc.subcore_barrier()

                in_d = [None] * NCH_S
                sc_d = [None] * NCH_S

                def fire_in(j):
                    d = pltpu.make_async_copy(
                        p_hbm.at[chg, pl.ds(ebase + j * CHUNK_S, CHUNK_S)],
                        pbuf.at[j % 2], sem_in)
                    d.start()
                    in_d[j] = d

                fire_in(0)
                for j in range(NCH_S):
                    if j >= 1:
                        sc_d[j - 1].wait()
                    if j + 1 < NCH_S:
                        fire_in(j + 1)
                    in_d[j].wait()
                    d = pltpu.make_async_copy(
                        pbuf.at[j % 2],
                        acc.at[segv.at[j, 0, pl.ds(0, CHUNK_S)]], sem_sc)
                    d.start(add=True)
                    sc_d[j] = d
                sc_d[NCH_S - 1].wait()
                plsc.subcore_barrier()
                pltpu.sync_copy(acc.at[pl.ds(rbase, ROWS_PT), 0],
                                out_hbm.at[chg, pl.ds(rbase, ROWS_PT)])
                plsc.subcore_barrier()


def _sc_scatter(payload, seg_r):
    mesh = plsc.VectorSubcoreMesh(core_axis_name="c", subcore_axis_name="s")
    zeros = jnp.zeros((ZROWS, 1, H), jnp.float32)
    return pl.kernel(
        _scatter_body,
        out_type=jax.ShapeDtypeStruct((NCH, NPAD, H), jnp.float32),
        mesh=mesh,
        scratch_types=[pltpu.VMEM((NCH_S, 1, CHUNK_G), jnp.int32),
                       pltpu.VMEM((2, CHUNK_S, 1, H), jnp.float32),
                       pltpu.VMEM((ZROWS, 1, H), jnp.float32),
                       pltpu.VMEM_SHARED((NPAD, 1, H), jnp.float32),
                       pltpu.SemaphoreType.DMA,
                       pltpu.SemaphoreType.DMA],
    )(payload, seg_r, zeros)


def _node_body(*refs):
    (*a_refs, lt0t_ref, lt1t_ref, lt2t_ref, ls1wt_ref, ls1b_ref,
     ls2wt_ref, ls2b_ref, lng_ref, lnb_ref, out_ref) = refs
    f32 = jnp.float32
    dot = functools.partial(jnp.dot, preferred_element_type=f32)

    def acc(k):
        v = a_refs[0][k]
        for a in a_refs[1:]:
            v = v + a[k]
        return v

    i_ = acc(0)
    ax = acc(1)
    ay = acc(2)
    az = acc(3)
    dxx = acc(4)
    dyy = acc(5)
    sxy = acc(6)
    sxz = acc(7)
    syz = acc(8)
    dzz = -dxx - dyy

    norm = (3.0 * i_ * i_
            + 2.0 * (ax * ax + ay * ay + az * az)
            + dxx * dxx + dyy * dyy + dzz * dzz
            + 2.0 * (sxy * sxy + sxz * sxz + syz * syz))
    mu = jnp.mean(norm, axis=1, keepdims=True)
    var = jnp.mean((norm - mu) ** 2, axis=1, keepdims=True)
    nrm = (norm - mu) * lax.rsqrt(var + 1e-5) * lng_ref[...] + lnb_ref[...]

    h1 = dot(nrm, ls1wt_ref[...]) + ls1b_ref[...]
    h1 = h1 * jax.nn.sigmoid(h1)
    h2 = dot(h1, ls2wt_ref[...]) + ls2b_ref[...]
    h2 = h2 * jax.nn.sigmoid(h2)
    n0 = h2[:, :H]
    n1 = h2[:, H:2 * H]
    n2 = h2[:, 2 * H:]

    lt2t = lt2t_ref[...]
    i2 = dot(i_, lt0t_ref[...])
    axp = dot(ax, lt1t_ref[...])
    ayp = dot(ay, lt1t_ref[...])
    azp = dot(az, lt1t_ref[...])
    dxxp = dot(dxx, lt2t)
    dyyp = dot(dyy, lt2t)
    dzzp = -dxxp - dyyp
    sxyp = dot(sxy, lt2t)
    sxzp = dot(sxz, lt2t)
    syzp = dot(syz, lt2t)

    diag = n0 * i2
    out_ref[0] = diag + n2 * dxxp
    out_ref[1] = -n1 * azp + n2 * sxyp
    out_ref[2] = n1 * ayp + n2 * sxzp
    out_ref[3] = n1 * azp + n2 * sxyp
    out_ref[4] = diag + n2 * dyyp
    out_ref[5] = -n1 * axp + n2 * syzp
    out_ref[6] = -n1 * ayp + n2 * sxzp
    out_ref[7] = n1 * axp + n2 * syzp
    out_ref[8] = diag + n2 * dzzp


def _tc_node(accs, lt0t, lt1t, lt2t, ls1wt, ls1br, ls2wt, ls2br,
             lngr, lnbr):
    full2 = lambda shape: pl.BlockSpec(shape, lambda i: (0, 0))
    aspec = pl.BlockSpec((NCH, NBLK, H), lambda i: (0, i, 0))
    return pl.pallas_call(
        _node_body,
        grid=(NPAD // NBLK,),
        in_specs=[aspec] * NSLICE + [
            full2((H, H)), full2((H, H)), full2((H, H)),
            full2((H, 2 * H)), full2((1, 2 * H)),
            full2((2 * H, 3 * H)), full2((1, 3 * H)),
            full2((1, H)), full2((1, H)),
        ],
        out_specs=pl.BlockSpec((9, NBLK, H), lambda i: (0, i, 0)),
        out_shape=jax.ShapeDtypeStruct((9, NPAD, H), jnp.float32),
    )(*accs, lt0t, lt1t, lt2t, ls1wt, ls1br, ls2wt, ls2br, lngr, lnbr)


def kernel(z, edge_index, edge_weight, edge_vec_norm, edge_attr,
           W1, b1, W2, b2, W3, b3, emb, emb2_W, emb2_b,
           lt0, lt1, lt2, ls1_W, ls1_b, ls2_W, ls2_b, ln_g, ln_b):
    f32 = jnp.float32
    i32 = jnp.int32
    z = z.astype(i32)
    ei = edge_index.astype(i32)
    pad_g = EPAD_G - E

    spread = jnp.arange(pad_g, dtype=i32) % N
    src_flat = jnp.concatenate([ei[0], spread])
    dst_flat = jnp.concatenate([ei[1], spread])
    z_pad = jnp.concatenate([z, jnp.zeros((NPAD - N,), i32)])

    seg4 = ei[0].reshape(NSLICE, NS, NCH_S, CHUNK_S)
    npadlanes = CHUNK_G - CHUNK_S
    dummy = N + (jnp.arange(NSLICE * NS * NCH_S * npadlanes, dtype=i32)
                 % (NPAD - N))
    seg_r = jnp.concatenate(
        [seg4, dummy.reshape(NSLICE, NS, NCH_S, npadlanes)],
        axis=3).reshape(NSLICE, NS, NCH_S, 1, CHUNK_G)

    w = edge_weight.astype(f32)
    cutoff = 0.5 * (jnp.cos(w * (jnp.pi / CUT_UP)) + 1.0)
    cutoff = cutoff * (w < CUT_UP).astype(f32)
    ev = edge_vec_norm.astype(f32)

    zsrc, zdst = _sc_gather(src_flat, dst_flat, z_pad)

    zero_e = jnp.zeros((E,), f32)
    misc = jnp.stack([cutoff, ev[:, 0], ev[:, 1], ev[:, 2],
                      zsrc[:E].astype(f32), zdst[:E].astype(f32),
                      zero_e, zero_e]).reshape(8, E // EB, EB) \
        .transpose(1, 0, 2)

    w123t = jnp.concatenate([W1.T, W2.T, W3.T], axis=1).astype(f32)
    b123 = jnp.concatenate([b1, b2, b3]).reshape(1, 3 * H).astype(f32)
    tcat = _tc_prep(emb.astype(f32), emb2_W[:, :H].T.astype(f32),
                    emb2_W[:, H:].T.astype(f32))
    embb = emb2_b.reshape(1, H)

    attr = edge_attr.astype(f32)
    accs = []
    for s in range(NSLICE):
        payload = _tc_edge(s, attr, misc, w123t, b123, tcat, embb)
        accs.append(_sc_scatter(payload, seg_r[s]))

    perm = (jnp.arange(3 * H) % 3) * H + (jnp.arange(3 * H) // 3)
    inv = jnp.argsort(perm)
    ls2_wg = ls2_W[inv]
    ls2_bg = ls2_b[inv]

    out9 = _tc_node(
        accs,
        lt0.T.astype(f32), lt1.T.astype(f32), lt2.T.astype(f32),
        ls1_W.T.astype(f32), ls1_b.reshape(1, 2 * H),
        ls2_wg.T.astype(f32), ls2_bg.reshape(1, 3 * H),
        ln_g.reshape(1, H), ln_b.reshape(1, H))

    return out9[:, :N].transpose(1, 2, 0).reshape(N, H, 3, 3)


# speedup vs baseline: 1.0276x; 1.0052x over previous
import functools

import jax
import jax.numpy as jnp
from jax import lax
from jax.experimental import pallas as pl
from jax.experimental.pallas import tpu as pltpu
from jax.experimental.pallas import tpu_sc as plsc

N = 5000
E = 80000
H = 128
NRBF = 64
MAXZ = 128
CUT_UP = 5.0

NPAD = 5120
NC = 2
NS = 16
NCH = 9

NSLICE = 2
ESL = E // NSLICE
EB = 1600
NB_SL = ESL // EB

EPAD_G = 81920
EPT_G = EPAD_G // (NC * NS)
CHUNK_G = 128

EPT_S = ESL // NS
CHUNK_S = 125
NCH_S = EPT_S // CHUNK_S
ROWS_PT = NPAD // NS
ZROWS = 64

NBLK = 512

SC_CH_BASE = (0, 5)
SC_CH_CNT = (5, 4)


def _gather_body(src_hbm, dst_hbm, z_hbm, zsrc_hbm, zdst_hbm,
                 eslab, oslab, sem):
    cid = lax.axis_index("c")
    sid = lax.axis_index("s")
    wid = cid * NS + sid
    base = wid * EPT_G
    nch = EPT_G // CHUNK_G

    def do_half(e_hbm, o_hbm):
        pltpu.sync_copy(e_hbm.at[wid], eslab)
        descs = []
        for j in range(nch):
            d = pltpu.make_async_copy(
                z_hbm.at[eslab.at[j]],
                oslab.at[pl.ds(j * CHUNK_G, CHUNK_G)], sem)
            d.start()
            descs.append(d)
        for d in descs:
            d.wait()
        pltpu.sync_copy(oslab, o_hbm.at[pl.ds(base, EPT_G)])

    do_half(src_hbm, zsrc_hbm)
    do_half(dst_hbm, zdst_hbm)


def _sc_gather(src_flat, dst_flat, z):
    mesh = plsc.VectorSubcoreMesh(core_axis_name="c", subcore_axis_name="s")
    nw = NC * NS
    src_r = src_flat.reshape(nw, EPT_G // CHUNK_G, CHUNK_G)
    dst_r = dst_flat.reshape(nw, EPT_G // CHUNK_G, CHUNK_G)
    return pl.kernel(
        _gather_body,
        out_type=[jax.ShapeDtypeStruct((EPAD_G,), jnp.int32),
                  jax.ShapeDtypeStruct((EPAD_G,), jnp.int32)],
        mesh=mesh,
        scratch_types=[pltpu.VMEM((EPT_G // CHUNK_G, CHUNK_G), jnp.int32),
                       pltpu.VMEM((EPT_G,), jnp.int32),
                       pltpu.SemaphoreType.DMA],
    )(src_r, dst_r, z)


def _prep_body(emb_ref, wlt_ref, wrt_ref, tcat_ref):
    f32 = jnp.float32
    tl = jnp.dot(emb_ref[...], wlt_ref[...], preferred_element_type=f32)
    tr = jnp.dot(emb_ref[...], wrt_ref[...], preferred_element_type=f32)
    tcat_ref[...] = jnp.concatenate([tl, tr], axis=0)


def _tc_prep(emb, wlt, wrt):
    return pl.pallas_call(
        _prep_body,
        out_shape=jax.ShapeDtypeStruct((2 * H, H), jnp.float32),
    )(emb, wlt, wrt)


def _edge_body(attr_ref, misc_ref, w123t_ref, b123_ref, tcat_ref, embb_ref,
               out_ref):
    f32 = jnp.float32
    mt = misc_ref[0].T
    c = mt[:, 0:1]
    vx = mt[:, 1:2]
    vy = mt[:, 2:3]
    vz = mt[:, 3:4]
    zs = mt[:, 4:5]
    zd = mt[:, 5:6]
    ioz = lax.broadcasted_iota(jnp.int32, (EB, MAXZ), 1).astype(f32)
    oh = jnp.concatenate([(zs == ioz), (zd == ioz)], axis=1).astype(f32)
    zij = (jnp.dot(oh, tcat_ref[...], preferred_element_type=f32)
           + embb_ref[...]) * c

    za = attr_ref[...]
    we = jnp.dot(za, w123t_ref[...], preferred_element_type=f32) \
        + b123_ref[...]
    m1 = zij * we[:, :H]
    m2 = zij * we[:, H:2 * H]
    m3 = zij * we[:, 2 * H:]
    r3 = (vx * vx + vy * vy + vz * vz) * (1.0 / 3.0)

    out_ref[0] = m1[:, None, :]
    out_ref[1] = (m2 * vx)[:, None, :]
    out_ref[2] = (m2 * vy)[:, None, :]
    out_ref[3] = (m2 * vz)[:, None, :]
    out_ref[4] = (m3 * (vx * vx - r3))[:, None, :]
    out_ref[5] = (m3 * (vy * vy - r3))[:, None, :]
    out_ref[6] = (m3 * (vx * vy))[:, None, :]
    out_ref[7] = (m3 * (vx * vz))[:, None, :]
    out_ref[8] = (m3 * (vy * vz))[:, None, :]


def _tc_edge(s, attr, misc, w123t, b123, tcat, embb):
    base = s * NB_SL
    full2 = lambda shape: pl.BlockSpec(shape, lambda i: (0, 0))
    return pl.pallas_call(
        _edge_body,
        grid=(NB_SL,),
        in_specs=[
            pl.BlockSpec((EB, NRBF), lambda i: (base + i, 0)),
            pl.BlockSpec((1, 8, EB), lambda i: (base + i, 0, 0)),
            full2((NRBF, 3 * H)), full2((1, 3 * H)),
            full2((2 * H, H)), full2((1, H)),
        ],
        out_specs=pl.BlockSpec((NCH, EB, 1, H), lambda i: (0, i, 0, 0)),
        out_shape=jax.ShapeDtypeStruct((NCH, ESL, 1, H), jnp.float32),
    )(attr, misc, w123t, b123, tcat, embb)


def _scatter_body(p_hbm, seg_hbm, zeros_hbm, out_hbm, segv, pbuf, zbuf, acc,
                  sem_in, sem_sc):
    cid = lax.axis_index("c")
    sid = lax.axis_index("s")
    ebase = sid * EPT_S
    rbase = sid * ROWS_PT
    pltpu.sync_copy(seg_hbm.at[sid], segv)
    pltpu.sync_copy(zeros_hbm, zbuf)

    for sc in range(NC):
        @pl.when(cid == sc)
        def _():
            for ch in range(SC_CH_CNT[sc]):
                chg = SC_CH_BASE[sc] + ch
                for k in range(ROWS_PT // ZROWS):
                    pltpu.sync_copy(
                        zbuf, acc.at[pl.ds(rbase + k * ZROWS, ZROWS)])
                plsc.subcore_barrier()

                in_d = [None] * NCH_S
                sc_d = [None] * NCH_S

                def fire_in(j):
                    d = pltpu.make_async_copy(
                        p_hbm.at[chg, pl.ds(ebase + j * CHUNK_S, CHUNK_S)],
                        pbuf.at[j % 2], sem_in)
                    d.start()
                    in_d[j] = d

                fire_in(0)
                for j in range(NCH_S):
                    if j >= 1:
                        sc_d[j - 1].wait()
                    if j + 1 < NCH_S:
                        fire_in(j + 1)
                    in_d[j].wait()
                    d = pltpu.make_async_copy(
                        pbuf.at[j % 2],
                        acc.at[segv.at[j, 0, pl.ds(0, CHUNK_S)]], sem_sc)
                    d.start(add=True)
                    sc_d[j] = d
                sc_d[NCH_S - 1].wait()
                plsc.subcore_barrier()
                pltpu.sync_copy(acc.at[pl.ds(rbase, ROWS_PT), 0],
                                out_hbm.at[chg, pl.ds(rbase, ROWS_PT)])
                plsc.subcore_barrier()


def _sc_scatter(payload, seg_r):
    mesh = plsc.VectorSubcoreMesh(core_axis_name="c", subcore_axis_name="s")
    zeros = jnp.zeros((ZROWS, 1, H), jnp.float32)
    return pl.kernel(
        _scatter_body,
        out_type=jax.ShapeDtypeStruct((NCH, NPAD, H), jnp.float32),
        mesh=mesh,
        scratch_types=[pltpu.VMEM((NCH_S, 1, CHUNK_G), jnp.int32),
                       pltpu.VMEM((2, CHUNK_S, 1, H), jnp.float32),
                       pltpu.VMEM((ZROWS, 1, H), jnp.float32),
                       pltpu.VMEM_SHARED((NPAD, 1, H), jnp.float32),
                       pltpu.SemaphoreType.DMA,
                       pltpu.SemaphoreType.DMA],
    )(payload, seg_r, zeros)


def _node_body(*refs):
    (*a_refs, lt0t_ref, lt1t_ref, lt2t_ref, ls1wt_ref, ls1b_ref,
     ls2wt_ref, ls2b_ref, lng_ref, lnb_ref, out_ref) = refs
    f32 = jnp.float32
    dot = functools.partial(jnp.dot, preferred_element_type=f32)

    def acc(k):
        v = a_refs[0][k]
        for a in a_refs[1:]:
            v = v + a[k]
        return v

    i_ = acc(0)
    ax = acc(1)
    ay = acc(2)
    az = acc(3)
    dxx = acc(4)
    dyy = acc(5)
    sxy = acc(6)
    sxz = acc(7)
    syz = acc(8)
    dzz = -dxx - dyy

    norm = (3.0 * i_ * i_
            + 2.0 * (ax * ax + ay * ay + az * az)
            + dxx * dxx + dyy * dyy + dzz * dzz
            + 2.0 * (sxy * sxy + sxz * sxz + syz * syz))
    mu = jnp.mean(norm, axis=1, keepdims=True)
    var = jnp.mean((norm - mu) ** 2, axis=1, keepdims=True)
    nrm = (norm - mu) * lax.rsqrt(var + 1e-5) * lng_ref[...] + lnb_ref[...]

    h1 = dot(nrm, ls1wt_ref[...]) + ls1b_ref[...]
    h1 = h1 * jax.nn.sigmoid(h1)
    h2 = dot(h1, ls2wt_ref[...]) + ls2b_ref[...]
    h2 = h2 * jax.nn.sigmoid(h2)
    n0 = h2[:, :H]
    n1 = h2[:, H:2 * H]
    n2 = h2[:, 2 * H:]

    lt2t = lt2t_ref[...]
    i2 = dot(i_, lt0t_ref[...])
    axp = dot(ax, lt1t_ref[...])
    ayp = dot(ay, lt1t_ref[...])
    azp = dot(az, lt1t_ref[...])
    dxxp = dot(dxx, lt2t)
    dyyp = dot(dyy, lt2t)
    dzzp = -dxxp - dyyp
    sxyp = dot(sxy, lt2t)
    sxzp = dot(sxz, lt2t)
    syzp = dot(syz, lt2t)

    diag = n0 * i2
    out_ref[0] = diag + n2 * dxxp
    out_ref[1] = -n1 * azp + n2 * sxyp
    out_ref[2] = n1 * ayp + n2 * sxzp
    out_ref[3] = n1 * azp + n2 * sxyp
    out_ref[4] = diag + n2 * dyyp
    out_ref[5] = -n1 * axp + n2 * syzp
    out_ref[6] = -n1 * ayp + n2 * sxzp
    out_ref[7] = n1 * axp + n2 * syzp
    out_ref[8] = diag + n2 * dzzp


def _tc_node(accs, lt0t, lt1t, lt2t, ls1wt, ls1br, ls2wt, ls2br,
             lngr, lnbr):
    full2 = lambda shape: pl.BlockSpec(shape, lambda i: (0, 0))
    aspec = pl.BlockSpec((NCH, NBLK, H), lambda i: (0, i, 0))
    return pl.pallas_call(
        _node_body,
        grid=(NPAD // NBLK,),
        in_specs=[aspec] * NSLICE + [
            full2((H, H)), full2((H, H)), full2((H, H)),
            full2((H, 2 * H)), full2((1, 2 * H)),
            full2((2 * H, 3 * H)), full2((1, 3 * H)),
            full2((1, H)), full2((1, H)),
        ],
        out_specs=pl.BlockSpec((9, NBLK, H), lambda i: (0, i, 0)),
        out_shape=jax.ShapeDtypeStruct((9, NPAD, H), jnp.float32),
    )(*accs, lt0t, lt1t, lt2t, ls1wt, ls1br, ls2wt, ls2br, lngr, lnbr)


def kernel(z, edge_index, edge_weight, edge_vec_norm, edge_attr,
           W1, b1, W2, b2, W3, b3, emb, emb2_W, emb2_b,
           lt0, lt1, lt2, ls1_W, ls1_b, ls2_W, ls2_b, ln_g, ln_b):
    f32 = jnp.float32
    i32 = jnp.int32
    z = z.astype(i32)
    ei = edge_index.astype(i32)
    pad_g = EPAD_G - E

    spread = jnp.arange(pad_g, dtype=i32) % N
    src_flat = jnp.concatenate([ei[0], spread])
    dst_flat = jnp.concatenate([ei[1], spread])
    z_pad = jnp.concatenate([z, jnp.zeros((NPAD - N,), i32)])

    seg4 = ei[0].reshape(NSLICE, NS, NCH_S, CHUNK_S)
    npadlanes = CHUNK_G - CHUNK_S
    dummy = N + (jnp.arange(NSLICE * NS * NCH_S * npadlanes, dtype=i32)
                 % (NPAD - N))
    seg_r = jnp.concatenate(
        [seg4, dummy.reshape(NSLICE, NS, NCH_S, npadlanes)],
        axis=3).reshape(NSLICE, NS, NCH_S, 1, CHUNK_G)

    w = edge_weight.astype(f32)
    cutoff = 0.5 * (jnp.cos(w * (jnp.pi / CUT_UP)) + 1.0)
    cutoff = cutoff * (w < CUT_UP).astype(f32)
    ev = edge_vec_norm.astype(f32)

    zsrc, zdst = _sc_gather(src_flat, dst_flat, z_pad)

    zero_e = jnp.zeros((E,), f32)
    misc = jnp.stack([cutoff, ev[:, 0], ev[:, 1], ev[:, 2],
                      zsrc[:E].astype(f32), zdst[:E].astype(f32),
                      zero_e, zero_e]).reshape(8, E // EB, EB) \
        .transpose(1, 0, 2)

    w123t = jnp.concatenate([W1.T, W2.T, W3.T], axis=1).astype(f32)
    b123 = jnp.concatenate([b1, b2, b3]).reshape(1, 3 * H).astype(f32)
    tcat = _tc_prep(emb.astype(f32), emb2_W[:, :H].T.astype(f32),
                    emb2_W[:, H:].T.astype(f32))
    embb = emb2_b.reshape(1, H)

    attr = edge_attr.astype(f32)
    accs = []
    for s in range(NSLICE):
        payload = _tc_edge(s, attr, misc, w123t, b123, tcat, embb)
        accs.append(_sc_scatter(payload, seg_r[s]))

    perm = (jnp.arange(3 * H) % 3) * H + (jnp.arange(3 * H) // 3)
    inv = jnp.argsort(perm)
    ls2_wg = ls2_W[inv]
    ls2_bg = ls2_b[inv]

    out9 = _tc_node(
        accs,
        lt0.T.astype(f32), lt1.T.astype(f32), lt2.T.astype(f32),
        ls1_W.T.astype(f32), ls1_b.reshape(1, 2 * H),
        ls2_wg.T.astype(f32), ls2_bg.reshape(1, 3 * H),
        ln_g.reshape(1, H), ln_b.reshape(1, H))

    return out9[:, :N].transpose(1, 2, 0).reshape(N, H, 3, 3)


# speedup vs baseline: 1.0614x; 1.0329x over previous
import functools

import jax
import jax.numpy as jnp
from jax import lax
from jax.experimental import pallas as pl
from jax.experimental.pallas import tpu as pltpu
from jax.experimental.pallas import tpu_sc as plsc

N = 5000
E = 80000
H = 128
NRBF = 64
MAXZ = 128
CUT_UP = 5.0

NPAD = 5120
NC = 2
NS = 16
NCH = 9

NSLICE = 2
ESL = E // NSLICE
EB = 1600
NB_SL = ESL // EB

EPAD_G = 81920
EPT_G = EPAD_G // (NC * NS)
CHUNK_G = 128

EPT_S = ESL // NS
CHUNK_S = 125
NCH_S = EPT_S // CHUNK_S
ROWS_PT = NPAD // NS
ZROWS = 64

NBLK = 512

SC_CH_BASE = (0, 5)
SC_CH_CNT = (5, 4)


def _gather_body(src_hbm, dst_hbm, z_hbm, zsrc_hbm, zdst_hbm,
                 eslab, oslab, sem):
    cid = lax.axis_index("c")
    sid = lax.axis_index("s")
    wid = cid * NS + sid
    base = wid * EPT_G
    nch = EPT_G // CHUNK_G

    def do_half(e_hbm, o_hbm):
        pltpu.sync_copy(e_hbm.at[wid], eslab)
        descs = []
        for j in range(nch):
            d = pltpu.make_async_copy(
                z_hbm.at[eslab.at[j]],
                oslab.at[pl.ds(j * CHUNK_G, CHUNK_G)], sem)
            d.start()
            descs.append(d)
        for d in descs:
            d.wait()
        pltpu.sync_copy(oslab, o_hbm.at[pl.ds(base, EPT_G)])

    do_half(src_hbm, zsrc_hbm)
    do_half(dst_hbm, zdst_hbm)


def _sc_gather(src_flat, dst_flat, z):
    mesh = plsc.VectorSubcoreMesh(core_axis_name="c", subcore_axis_name="s")
    nw = NC * NS
    src_r = src_flat.reshape(nw, EPT_G // CHUNK_G, CHUNK_G)
    dst_r = dst_flat.reshape(nw, EPT_G // CHUNK_G, CHUNK_G)
    return pl.kernel(
        _gather_body,
        out_type=[jax.ShapeDtypeStruct((EPAD_G,), jnp.int32),
                  jax.ShapeDtypeStruct((EPAD_G,), jnp.int32)],
        mesh=mesh,
        scratch_types=[pltpu.VMEM((EPT_G // CHUNK_G, CHUNK_G), jnp.int32),
                       pltpu.VMEM((EPT_G,), jnp.int32),
                       pltpu.SemaphoreType.DMA],
    )(src_r, dst_r, z)


def _prep_body(emb_ref, wlt_ref, wrt_ref, tcat_ref):
    f32 = jnp.float32
    tl = jnp.dot(emb_ref[...], wlt_ref[...], preferred_element_type=f32)
    tr = jnp.dot(emb_ref[...], wrt_ref[...], preferred_element_type=f32)
    tcat_ref[...] = jnp.concatenate([tl, tr], axis=0)


def _tc_prep(emb, wlt, wrt):
    return pl.pallas_call(
        _prep_body,
        out_shape=jax.ShapeDtypeStruct((2 * H, H), jnp.float32),
    )(emb, wlt, wrt)


def _edge_body(attr_ref, misc_ref, w123t_ref, b123_ref, tcat_ref, embb_ref,
               out_ref):
    f32 = jnp.float32
    mt = misc_ref[0].T
    c = mt[:, 0:1]
    vx = mt[:, 1:2]
    vy = mt[:, 2:3]
    vz = mt[:, 3:4]
    zs = mt[:, 4:5]
    zd = mt[:, 5:6]
    ioz = lax.broadcasted_iota(jnp.int32, (EB, MAXZ), 1).astype(f32)
    oh = jnp.concatenate([(zs == ioz), (zd == ioz)], axis=1).astype(f32)
    zij = (jnp.dot(oh, tcat_ref[...], preferred_element_type=f32)
           + embb_ref[...]) * c

    za = attr_ref[...]
    we = jnp.dot(za, w123t_ref[...], preferred_element_type=f32) \
        + b123_ref[...]
    m1 = zij * we[:, :H]
    m2 = zij * we[:, H:2 * H]
    m3 = zij * we[:, 2 * H:]
    r3 = (vx * vx + vy * vy + vz * vz) * (1.0 / 3.0)

    out_ref[0] = m1[:, None, :]
    out_ref[1] = (m2 * vx)[:, None, :]
    out_ref[2] = (m2 * vy)[:, None, :]
    out_ref[3] = (m2 * vz)[:, None, :]
    out_ref[4] = (m3 * (vx * vx - r3))[:, None, :]
    out_ref[5] = (m3 * (vy * vy - r3))[:, None, :]
    out_ref[6] = (m3 * (vx * vy))[:, None, :]
    out_ref[7] = (m3 * (vx * vz))[:, None, :]
    out_ref[8] = (m3 * (vy * vz))[:, None, :]


def _tc_edge(s, attr, misc, w123t, b123, tcat, embb):
    base = s * NB_SL
    full2 = lambda shape: pl.BlockSpec(shape, lambda i: (0, 0))
    return pl.pallas_call(
        _edge_body,
        grid=(NB_SL,),
        in_specs=[
            pl.BlockSpec((EB, NRBF), lambda i: (base + i, 0)),
            pl.BlockSpec((1, 8, EB), lambda i: (base + i, 0, 0)),
            full2((NRBF, 3 * H)), full2((1, 3 * H)),
            full2((2 * H, H)), full2((1, H)),
        ],
        out_specs=pl.BlockSpec((NCH, EB, 1, H), lambda i: (0, i, 0, 0)),
        out_shape=jax.ShapeDtypeStruct((NCH, ESL, 1, H), jnp.float32),
    )(attr, misc, w123t, b123, tcat, embb)


def _scatter_body(p_hbm, seg_hbm, zeros_hbm, out_hbm, segv, pbuf, zbuf,
                  acc_a, acc_b, sem_in, sem_sc, sem_d):
    cid = lax.axis_index("c")
    sid = lax.axis_index("s")
    ebase = sid * EPT_S
    rbase = sid * ROWS_PT
    pltpu.sync_copy(seg_hbm.at[sid], segv)
    pltpu.sync_copy(zeros_hbm, zbuf)
    accs = (acc_a, acc_b)

    def zero_rows(acc):
        for k in range(ROWS_PT // ZROWS):
            pltpu.sync_copy(zbuf, acc.at[pl.ds(rbase + k * ZROWS, ZROWS)])

    def do_channel(chg, acc, mid):
        in_d = [None] * NCH_S
        sc_d = [None] * NCH_S

        def fire_in(j):
            d = pltpu.make_async_copy(
                p_hbm.at[chg, pl.ds(ebase + j * CHUNK_S, CHUNK_S)],
                pbuf.at[j % 2], sem_in)
            d.start()
            in_d[j] = d

        fire_in(0)
        for j in range(NCH_S):
            if j >= 1:
                sc_d[j - 1].wait()
            if j + 1 < NCH_S:
                fire_in(j + 1)
            if j == 3 and mid is not None:
                mid()
            in_d[j].wait()
            d = pltpu.make_async_copy(
                pbuf.at[j % 2],
                acc.at[segv.at[j, 0, pl.ds(0, CHUNK_S)]], sem_sc)
            d.start(add=True)
            sc_d[j] = d
        sc_d[NCH_S - 1].wait()

    for sc in range(NC):
        @pl.when(cid == sc)
        def _():
            ncsc = SC_CH_CNT[sc]
            zero_rows(accs[0])
            plsc.subcore_barrier()
            dump_d = [None]
            for ch in range(ncsc):
                chg = SC_CH_BASE[sc] + ch
                cur = accs[ch % 2]
                prev = accs[1 - ch % 2]
                if ch > 0:
                    d = pltpu.make_async_copy(
                        prev.at[pl.ds(rbase, ROWS_PT), 0],
                        out_hbm.at[chg - 1, pl.ds(rbase, ROWS_PT)], sem_d)
                    d.start()
                    dump_d[0] = d

                if ch == 0 and ncsc > 1:
                    def mid():
                        zero_rows(accs[1])
                elif ch > 0 and ch + 1 < ncsc:
                    def mid():
                        dump_d[0].wait()
                        zero_rows(prev)
                else:
                    mid = None

                do_channel(chg, cur, mid)
                if ch > 0 and ch + 1 >= ncsc:
                    dump_d[0].wait()
                plsc.subcore_barrier()
            last = accs[(ncsc - 1) % 2]
            pltpu.sync_copy(
                last.at[pl.ds(rbase, ROWS_PT), 0],
                out_hbm.at[SC_CH_BASE[sc] + ncsc - 1, pl.ds(rbase, ROWS_PT)])


def _sc_scatter(payload, seg_r):
    mesh = plsc.VectorSubcoreMesh(core_axis_name="c", subcore_axis_name="s")
    zeros = jnp.zeros((ZROWS, 1, H), jnp.float32)
    return pl.kernel(
        _scatter_body,
        out_type=jax.ShapeDtypeStruct((NCH, NPAD, H), jnp.float32),
        mesh=mesh,
        scratch_types=[pltpu.VMEM((NCH_S, 1, CHUNK_G), jnp.int32),
                       pltpu.VMEM((2, CHUNK_S, 1, H), jnp.float32),
                       pltpu.VMEM((ZROWS, 1, H), jnp.float32),
                       pltpu.VMEM_SHARED((NPAD, 1, H), jnp.float32),
                       pltpu.VMEM_SHARED((NPAD, 1, H), jnp.float32),
                       pltpu.SemaphoreType.DMA,
                       pltpu.SemaphoreType.DMA,
                       pltpu.SemaphoreType.DMA],
    )(payload, seg_r, zeros)


def _node_body(*refs):
    (*a_refs, lt0t_ref, lt1t_ref, lt2t_ref, ls1wt_ref, ls1b_ref,
     ls2wt_ref, ls2b_ref, lng_ref, lnb_ref, out_ref) = refs
    f32 = jnp.float32
    dot = functools.partial(jnp.dot, preferred_element_type=f32)

    def acc(k):
        v = a_refs[0][k]
        for a in a_refs[1:]:
            v = v + a[k]
        return v

    i_ = acc(0)
    ax = acc(1)
    ay = acc(2)
    az = acc(3)
    dxx = acc(4)
    dyy = acc(5)
    sxy = acc(6)
    sxz = acc(7)
    syz = acc(8)
    dzz = -dxx - dyy

    norm = (3.0 * i_ * i_
            + 2.0 * (ax * ax + ay * ay + az * az)
            + dxx * dxx + dyy * dyy + dzz * dzz
            + 2.0 * (sxy * sxy + sxz * sxz + syz * syz))
    mu = jnp.mean(norm, axis=1, keepdims=True)
    var = jnp.mean((norm - mu) ** 2, axis=1, keepdims=True)
    nrm = (norm - mu) * lax.rsqrt(var + 1e-5) * lng_ref[...] + lnb_ref[...]

    h1 = dot(nrm, ls1wt_ref[...]) + ls1b_ref[...]
    h1 = h1 * jax.nn.sigmoid(h1)
    h2 = dot(h1, ls2wt_ref[...]) + ls2b_ref[...]
    h2 = h2 * jax.nn.sigmoid(h2)
    n0 = h2[:, :H]
    n1 = h2[:, H:2 * H]
    n2 = h2[:, 2 * H:]

    lt2t = lt2t_ref[...]
    i2 = dot(i_, lt0t_ref[...])
    axp = dot(ax, lt1t_ref[...])
    ayp = dot(ay, lt1t_ref[...])
    azp = dot(az, lt1t_ref[...])
    dxxp = dot(dxx, lt2t)
    dyyp = dot(dyy, lt2t)
    dzzp = -dxxp - dyyp
    sxyp = dot(sxy, lt2t)
    sxzp = dot(sxz, lt2t)
    syzp = dot(syz, lt2t)

    diag = n0 * i2
    out_ref[0] = diag + n2 * dxxp
    out_ref[1] = -n1 * azp + n2 * sxyp
    out_ref[2] = n1 * ayp + n2 * sxzp
    out_ref[3] = n1 * azp + n2 * sxyp
    out_ref[4] = diag + n2 * dyyp
    out_ref[5] = -n1 * axp + n2 * syzp
    out_ref[6] = -n1 * ayp + n2 * sxzp
    out_ref[7] = n1 * axp + n2 * syzp
    out_ref[8] = diag + n2 * dzzp


def _tc_node(accs, lt0t, lt1t, lt2t, ls1wt, ls1br, ls2wt, ls2br,
             lngr, lnbr):
    full2 = lambda shape: pl.BlockSpec(shape, lambda i: (0, 0))
    aspec = pl.BlockSpec((NCH, NBLK, H), lambda i: (0, i, 0))
    return pl.pallas_call(
        _node_body,
        grid=(NPAD // NBLK,),
        in_specs=[aspec] * NSLICE + [
            full2((H, H)), full2((H, H)), full2((H, H)),
            full2((H, 2 * H)), full2((1, 2 * H)),
            full2((2 * H, 3 * H)), full2((1, 3 * H)),
            full2((1, H)), full2((1, H)),
        ],
        out_specs=pl.BlockSpec((9, NBLK, H), lambda i: (0, i, 0)),
        out_shape=jax.ShapeDtypeStruct((9, NPAD, H), jnp.float32),
    )(*accs, lt0t, lt1t, lt2t, ls1wt, ls1br, ls2wt, ls2br, lngr, lnbr)


def kernel(z, edge_index, edge_weight, edge_vec_norm, edge_attr,
           W1, b1, W2, b2, W3, b3, emb, emb2_W, emb2_b,
           lt0, lt1, lt2, ls1_W, ls1_b, ls2_W, ls2_b, ln_g, ln_b):
    f32 = jnp.float32
    i32 = jnp.int32
    z = z.astype(i32)
    ei = edge_index.astype(i32)
    pad_g = EPAD_G - E

    spread = jnp.arange(pad_g, dtype=i32) % N
    src_flat = jnp.concatenate([ei[0], spread])
    dst_flat = jnp.concatenate([ei[1], spread])
    z_pad = jnp.concatenate([z, jnp.zeros((NPAD - N,), i32)])

    seg4 = ei[0].reshape(NSLICE, NS, NCH_S, CHUNK_S)
    npadlanes = CHUNK_G - CHUNK_S
    dummy = N + (jnp.arange(NSLICE * NS * NCH_S * npadlanes, dtype=i32)
                 % (NPAD - N))
    seg_r = jnp.concatenate(
        [seg4, dummy.reshape(NSLICE, NS, NCH_S, npadlanes)],
        axis=3).reshape(NSLICE, NS, NCH_S, 1, CHUNK_G)

    w = edge_weight.astype(f32)
    cutoff = 0.5 * (jnp.cos(w * (jnp.pi / CUT_UP)) + 1.0)
    cutoff = cutoff * (w < CUT_UP).astype(f32)
    ev = edge_vec_norm.astype(f32)

    zsrc, zdst = _sc_gather(src_flat, dst_flat, z_pad)

    zero_e = jnp.zeros((E,), f32)
    misc = jnp.stack([cutoff, ev[:, 0], ev[:, 1], ev[:, 2],
                      zsrc[:E].astype(f32), zdst[:E].astype(f32),
                      zero_e, zero_e]).reshape(8, E // EB, EB) \
        .transpose(1, 0, 2)

    w123t = jnp.concatenate([W1.T, W2.T, W3.T], axis=1).astype(f32)
    b123 = jnp.concatenate([b1, b2, b3]).reshape(1, 3 * H).astype(f32)
    tcat = _tc_prep(emb.astype(f32), emb2_W[:, :H].T.astype(f32),
                    emb2_W[:, H:].T.astype(f32))
    embb = emb2_b.reshape(1, H)

    attr = edge_attr.astype(f32)
    accs = []
    for s in range(NSLICE):
        payload = _tc_edge(s, attr, misc, w123t, b123, tcat, embb)
        accs.append(_sc_scatter(payload, seg_r[s]))

    perm = (jnp.arange(3 * H) % 3) * H + (jnp.arange(3 * H) // 3)
    inv = jnp.argsort(perm)
    ls2_wg = ls2_W[inv]
    ls2_bg = ls2_b[inv]

    out9 = _tc_node(
        accs,
        lt0.T.astype(f32), lt1.T.astype(f32), lt2.T.astype(f32),
        ls1_W.T.astype(f32), ls1_b.reshape(1, 2 * H),
        ls2_wg.T.astype(f32), ls2_bg.reshape(1, 3 * H),
        ln_g.reshape(1, H), ln_b.reshape(1, H))

    return out9[:, :N].transpose(1, 2, 0).reshape(N, H, 3, 3)
